# baseline (device time: 97537 ns/iter reference)
import jax
import jax.numpy as jnp
from jax import lax
from jax.experimental import pallas as pl
from jax.experimental.pallas import tpu as pltpu

N_DEV = 32
N_STEPS = 5
B, S, D = 2, 256, 1024
H, Dh, Dr = 16, 64, 32
ROWS = B * S
KV_COLS = 2 * D
N_STREAMS = 2
SROWS = ROWS // N_STREAMS

_AXES = {
    "x": (1, 0, 0),
    "y1": (0, 1, 0),
    "y2": (0, 2, 0),
    "z1": (0, 0, 1),
    "z2": (0, 0, 2),
}
_ORDERS = [
    ["x", "y1", "z1", "z2", "y2"],
    ["y1", "z1", "x", "y2", "z2"],
]


def _coords(my):
    z = my >> 3
    q = my & 7
    y = q >> 1
    x = (q & 1) ^ (y & 1)
    return x, y, z


def _partner_and_bit(my, axis):
    fx, fy, fz = _AXES[axis]
    x, y, z = _coords(my)
    px, py, pz = x ^ fx, y ^ fy, z ^ fz
    ppos = (pz << 3) + (py << 1) + ((px ^ (py & 1)) & 1)
    if axis == "x":
        bit = x
    elif axis == "y1":
        bit = y & 1
    elif axis == "y2":
        bit = (y >> 1) & 1
    elif axis == "z1":
        bit = z & 1
    else:
        bit = (z >> 1) & 1
    return ppos, bit


def _allreduce_kv_body(x_ref, wdkv_ref, wuk_ref, wuv_ref,
                       wq_ref, wqr_ref, wkr_ref,
                       kv_ref, q_ref, qr_ref, kr_ref,
                       *scratch):
    slots = scratch[0:N_STEPS]
    rs_send, rs_recv, ag_send, ag_recv = scratch[N_STEPS:]
    my = lax.axis_index("i")

    x2 = x_ref[...].reshape(ROWS, D)
    c = jnp.dot(x2, wdkv_ref[...], preferred_element_type=jnp.float32)
    kv_ref[:, 0:D] = jnp.dot(c, wuk_ref[...],
                             preferred_element_type=jnp.float32)
    kv_ref[:, D:KV_COLS] = jnp.dot(c, wuv_ref[...],
                                   preferred_element_type=jnp.float32)

    barrier_sem = pltpu.get_barrier_semaphore()
    for axis in _AXES:
        partner, _ = _partner_and_bit(my, axis)
        pl.semaphore_signal(barrier_sem, inc=1, device_id=(partner,),
                            device_id_type=pl.DeviceIdType.MESH)
    pl.semaphore_wait(barrier_sem, N_STEPS)

    lo = [jnp.int32(st * SROWS) for st in range(N_STREAMS)]
    n = [SROWS] * N_STREAMS
    for step in range(N_STEPS):
        rdmas = []
        for st in range(N_STREAMS):
            partner, bit = _partner_and_bit(my, _ORDERS[st][step])
            half = n[st] // 2
            keep_lo = lo[st] + bit * half
            send_lo = lo[st] + (1 - bit) * half
            rdma = pltpu.make_async_remote_copy(
                src_ref=kv_ref.at[pl.ds(send_lo, half), :],
                dst_ref=slots[step].at[st],
                send_sem=rs_send.at[step, st],
                recv_sem=rs_recv.at[step, st],
                device_id=(partner,),
                device_id_type=pl.DeviceIdType.MESH,
            )
            rdma.start()
            rdmas.append(rdma)
            lo[st] = keep_lo
            n[st] = half
        if step == 0:
            q_ref[...] = jnp.dot(x2, wq_ref[...],
                                 preferred_element_type=jnp.float32)
            qr_ref[...] = jnp.dot(x2, wqr_ref[...],
                                  preferred_element_type=jnp.float32)
            kr_ref[...] = jnp.dot(x2, wkr_ref[...],
                                  preferred_element_type=jnp.float32)
        for st in range(N_STREAMS):
            rdmas[st].wait()
            kv_ref[pl.ds(lo[st], n[st]), :] = (
                kv_ref[pl.ds(lo[st], n[st]), :] + slots[step][st]
            )

    for step in reversed(range(N_STEPS)):
        rdmas = []
        for st in range(N_STREAMS):
            partner, bit = _partner_and_bit(my, _ORDERS[st][step])
            rdma = pltpu.make_async_remote_copy(
                src_ref=kv_ref.at[pl.ds(lo[st], n[st]), :],
                dst_ref=kv_ref.at[pl.ds(lo[st], n[st]), :],
                send_sem=ag_send.at[step, st],
                recv_sem=ag_recv.at[step, st],
                device_id=(partner,),
                device_id_type=pl.DeviceIdType.MESH,
            )
            rdma.start()
            rdmas.append(rdma)
            lo[st] = lo[st] - bit * n[st]
            n[st] = 2 * n[st]
        for st in range(N_STREAMS):
            rdmas[st].wait()


def _allreduce_kv(x, Wdkv, Wuk, Wuv, Wq, Wqr, Wkr):
    return pl.pallas_call(
        _allreduce_kv_body,
        out_shape=[
            jax.ShapeDtypeStruct((ROWS, KV_COLS), jnp.float32),
            jax.ShapeDtypeStruct((ROWS, H * Dh), jnp.float32),
            jax.ShapeDtypeStruct((ROWS, H * Dr), jnp.float32),
            jax.ShapeDtypeStruct((ROWS, Dr), jnp.float32),
        ],
        in_specs=[pl.BlockSpec(memory_space=pltpu.VMEM)] * 7,
        out_specs=[pl.BlockSpec(memory_space=pltpu.VMEM)] * 4,
        scratch_shapes=(
            [pltpu.VMEM((N_STREAMS, SROWS >> (s + 1), KV_COLS), jnp.float32)
             for s in range(N_STEPS)]
            + [pltpu.SemaphoreType.DMA((N_STEPS, N_STREAMS))] * 4
        ),
        compiler_params=pltpu.CompilerParams(collective_id=0),
    )(x, Wdkv, Wuk, Wuv, Wq, Wqr, Wkr)


def kernel(x, Wdkv, Wuk, Wuv, Wq, Wqr, Wkr, Wo):
    kv, q, qr, kr = _allreduce_kv(x, Wdkv, Wuk, Wuv, Wq, Wqr, Wkr)
    K = kv[:, :D].reshape(B, S, H, Dh)
    V = kv[:, D:].reshape(B, S, H, Dh)
    Q = q.reshape(B, S, H, Dh)
    Qr = qr.reshape(B, S, H, Dr)
    Kr = kr.reshape(B, S, Dr)
    scale = (Dh + Dr) ** -0.5
    scores = (jnp.einsum("bshd,bthd->bhst", Q, K)
              + jnp.einsum("bshd,btd->bhst", Qr, Kr)) * scale
    m = scores.max(-1, keepdims=True)
    p = jnp.exp(scores - m)
    p = p / p.sum(-1, keepdims=True)
    O = jnp.einsum("bhst,bthd->bshd", p, V).reshape(ROWS, H * Dh)
    return (O @ Wo).reshape(B, S, D)
